# baseline (speedup 1.0000x reference)
import functools
import numpy as np
import jax
import jax.numpy as jnp
from jax import lax
from jax.experimental import pallas as pl
from jax.experimental.pallas import tpu as pltpu
from jax.experimental.pallas import tpu_sc as plsc

DIM = 32
SCALE = np.sqrt(np.float32(DIM)).astype(np.float32)
NC, NS = 2, 16
NW = NC * NS
NB2 = 2
NBUF = 4
GATHER_AHEAD = 3


@functools.lru_cache(maxsize=None)
def _make_transpose(V):
    tcols = V // 128
    per_w = tcols // NW
    rem = tcols % NW
    tail = V - tcols * 128
    ngroups = per_w // NBUF
    assert per_w % NBUF == 0 and tail % 16 == 0
    mesh = plsc.VectorSubcoreMesh(
        core_axis_name="c", subcore_axis_name="s",
        num_cores=NC, num_subcores=NS)

    @functools.partial(
        pl.kernel,
        out_type=jax.ShapeDtypeStruct((V * DIM // 128, 128), jnp.float32),
        mesh=mesh,
        scratch_types=(
            [pltpu.VMEM((DIM, 131), jnp.float32)] * NBUF
            + [pltpu.VMEM((DIM, 128), jnp.float32)] * NBUF
            + [pltpu.SemaphoreType.DMA] * (2 * NBUF)
        ),
        compiler_params=pltpu.CompilerParams(
            use_tc_tiling_on_sc=True, needs_layout_passes=False),
    )
    def tr_kernel(tt_hbm, out_hbm, *scratch):
        ibufs = scratch[:NBUF]
        obufs = scratch[NBUF:2 * NBUF]
        lsems = scratch[2 * NBUF:3 * NBUF]
        ssems = scratch[3 * NBUF:]
        wid = lax.axis_index("s") * NC + lax.axis_index("c")
        blk0 = wid * per_w

        lane = lax.iota(jnp.int32, 16)
        drows = [lane + 16 * (k % 2) for k in range(8)]

        def fire_load(c, b):
            pltpu.async_copy(
                tt_hbm.at[:, pl.ds((blk0 + c) * 128, 128)],
                ibufs[b].at[:, pl.ds(0, 128)], lsems[b])

        def wait_load(b):
            pltpu.make_async_copy(
                tt_hbm.at[:, pl.ds(0, 128)],
                ibufs[b].at[:, pl.ds(0, 128)], lsems[b]).wait()

        def fire_scatter(c, b):
            pltpu.async_copy(
                obufs[b], out_hbm.at[pl.ds((blk0 + c) * DIM, DIM)], ssems[b])

        def wait_scatter(b):
            pltpu.make_async_copy(
                obufs[b], out_hbm.at[pl.ds(0, DIM)], ssems[b]).wait()

        def transpose(b, nrows=DIM):
            ibuf, obuf = ibufs[b], obufs[b]

            @plsc.parallel_loop(0, nrows, step=1, unroll=4)
            def _tr(i):
                for k in range(8):
                    vcol = jnp.broadcast_to(4 * i + k // 2, (16,))
                    vec = plsc.load_gather(ibuf, [drows[k], vcol])
                    obuf[i, pl.ds(16 * k, 16)] = vec * SCALE

        for c in range(GATHER_AHEAD):
            fire_load(c, c % NBUF)

        @pl.loop(0, ngroups)
        def _group(g):
            for i in range(NBUF):
                c = g * NBUF + i
                wait_load(i)

                @pl.when(c >= NBUF)
                def _():
                    wait_scatter(i)

                transpose(i)
                fire_scatter(c, i)

                @pl.when(c + GATHER_AHEAD < per_w)
                def _():
                    fire_load(c + GATHER_AHEAD, (i + GATHER_AHEAD) % NBUF)

        for c in range(per_w - NBUF, per_w):
            wait_scatter(c % NBUF)

        @pl.when(wid < rem)
        def _():
            blk = tcols - rem + wid
            pltpu.sync_copy(tt_hbm.at[:, pl.ds(blk * 128, 128)],
                            ibufs[0].at[:, pl.ds(0, 128)])
            transpose(0)
            pltpu.sync_copy(obufs[0], out_hbm.at[pl.ds(blk * DIM, DIM)])

    return tr_kernel


@functools.lru_cache(maxsize=None)
def _make(B1, B2):
    cols_w = B1 // NW
    n_chunks = B2 // NB2
    n_groups = n_chunks // NBUF
    assert B2 % NB2 == 0 and n_chunks % NBUF == 0
    mesh = plsc.VectorSubcoreMesh(
        core_axis_name="c", subcore_axis_name="s",
        num_cores=NC, num_subcores=NS)

    @functools.partial(
        pl.kernel,
        out_type=jax.ShapeDtypeStruct((B2, DIM // 8, B1 // 128, 8, 128),
                                      jnp.float32),
        mesh=mesh,
        scratch_types=(
            [pltpu.VMEM((B2, cols_w), jnp.int32)]
            + [pltpu.VMEM((NB2 * cols_w, DIM), jnp.float32)] * NBUF
            + [pltpu.VMEM((NB2, DIM // 8, 8, cols_w + 1), jnp.float32)] * NBUF
            + [pltpu.SemaphoreType.DMA] * (2 * NBUF)
        ),
        compiler_params=pltpu.CompilerParams(
            use_tc_tiling_on_sc=False, needs_layout_passes=False),
    )
    def emb_kernel(table_hbm, xt_hbm, out_hbm, idx_v, *scratch):
        gbufs = scratch[:NBUF]
        obufs = scratch[NBUF:2 * NBUF]
        gsems = scratch[2 * NBUF:3 * NBUF]
        ssems = scratch[3 * NBUF:]
        wid = lax.axis_index("s") * NC + lax.axis_index("c")
        col0 = wid * cols_w

        def fire_gather(c, b):
            for s in range(NB2):
                pltpu.async_copy(
                    table_hbm.at[idx_v.at[c * NB2 + s]],
                    gbufs[b].at[pl.ds(s * cols_w, cols_w)],
                    gsems[b])

        def wait_gather(b):
            pltpu.make_async_copy(
                table_hbm.at[pl.ds(0, NB2 * cols_w)],
                gbufs[b], gsems[b]).wait()

        def fire_scatter(c, b):
            pltpu.async_copy(
                obufs[b].at[:, :, :, pl.ds(0, cols_w)],
                out_hbm.at[pl.ds(c * NB2, NB2), :, wid],
                ssems[b])

        def wait_scatter(b):
            pltpu.make_async_copy(
                obufs[b].at[:, :, :, pl.ds(0, cols_w)],
                out_hbm.at[pl.ds(0, NB2), :, 0],
                ssems[b]).wait()

        lane = lax.iota(jnp.int32, 16)
        dim_rows = [lane + 16 * h for h in range(DIM // 16)]
        big_rows = [lax.div(d, 8) for d in dim_rows]
        sub_rows = [lax.rem(d, 8) for d in dim_rows]
        s_ids = [jnp.broadcast_to(jnp.int32(s), (16,)) for s in range(NB2)]

        def scale(b):
            gbuf, obuf = gbufs[b], obufs[b]

            @plsc.parallel_loop(0, cols_w, step=1, unroll=4)
            def _scale(l):
                lcol = jnp.broadcast_to(l, (16,))
                for s in range(NB2):
                    for h in range(DIM // 16):
                        vec = gbuf[s * cols_w + l, pl.ds(16 * h, 16)]
                        plsc.store_scatter(
                            obuf, [s_ids[s], big_rows[h], sub_rows[h], lcol],
                            vec)

        pltpu.sync_copy(xt_hbm.at[:, pl.ds(col0, cols_w)], idx_v)

        for c in range(GATHER_AHEAD):
            fire_gather(c, c % NBUF)

        @pl.loop(0, n_groups)
        def _group(g):
            for i in range(NBUF):
                c = g * NBUF + i
                wait_gather(i)

                @pl.when(c >= NBUF)
                def _():
                    wait_scatter(i)

                scale(i)
                fire_scatter(c, i)

                @pl.when(c + GATHER_AHEAD < n_chunks)
                def _():
                    fire_gather(c + GATHER_AHEAD, (i + GATHER_AHEAD) % NBUF)

        for c in range(n_chunks - NBUF, n_chunks):
            wait_scatter(c % NBUF)

    return emb_kernel


def kernel(x, table):
    B1, B2 = x.shape
    V = table.shape[0]
    xt = jnp.transpose(x, (1, 0)).astype(jnp.int32)
    tt = jnp.transpose(table, (1, 0))
    t = _make_transpose(V)(tt).reshape(V, DIM)
    vtail = (V // 128) * 128
    t = t.at[vtail:].set(table[vtail:] * SCALE)
    out5 = _make(B1, B2)(t, xt)
    out = jnp.transpose(out5, (0, 1, 3, 2, 4)).reshape(B2, DIM, B1)
    return jnp.transpose(out, (2, 0, 1))


# speedup vs baseline: 1.9981x; 1.9981x over previous
import functools
import numpy as np
import jax
import jax.numpy as jnp
from jax import lax
from jax.experimental import pallas as pl
from jax.experimental.pallas import tpu as pltpu
from jax.experimental.pallas import tpu_sc as plsc

DIM = 32
SCALE = np.sqrt(np.float32(DIM)).astype(np.float32)
NC, NS = 2, 16
NW = NC * NS
NB2 = 2
NBUF = 4
GATHER_AHEAD = 3


@functools.lru_cache(maxsize=None)
def _make(B1, B2):
    cols_w = B1 // NW
    n_chunks = B2 // NB2
    n_groups = n_chunks // NBUF
    assert B2 % NB2 == 0 and n_chunks % NBUF == 0
    mesh = plsc.VectorSubcoreMesh(
        core_axis_name="c", subcore_axis_name="s",
        num_cores=NC, num_subcores=NS)

    @functools.partial(
        pl.kernel,
        out_type=jax.ShapeDtypeStruct((B2, DIM // 8, B1 // 128, 8, 128),
                                      jnp.float32),
        mesh=mesh,
        scratch_types=(
            [pltpu.VMEM((B2, cols_w), jnp.int32)]
            + [pltpu.VMEM((NB2 * cols_w, DIM), jnp.float32)] * NBUF
            + [pltpu.VMEM((NB2, DIM // 8, 8, cols_w + 1), jnp.float32)] * NBUF
            + [pltpu.SemaphoreType.DMA] * (2 * NBUF)
        ),
        compiler_params=pltpu.CompilerParams(
            use_tc_tiling_on_sc=False, needs_layout_passes=False),
    )
    def emb_kernel(table_hbm, xt_hbm, out_hbm, idx_v, *scratch):
        gbufs = scratch[:NBUF]
        obufs = scratch[NBUF:2 * NBUF]
        gsems = scratch[2 * NBUF:3 * NBUF]
        ssems = scratch[3 * NBUF:]
        wid = lax.axis_index("s") * NC + lax.axis_index("c")
        col0 = wid * cols_w

        def fire_gather(c, b):
            for s in range(NB2):
                pltpu.async_copy(
                    table_hbm.at[idx_v.at[c * NB2 + s]],
                    gbufs[b].at[pl.ds(s * cols_w, cols_w)],
                    gsems[b])

        def wait_gather(b):
            pltpu.make_async_copy(
                table_hbm.at[pl.ds(0, NB2 * cols_w)],
                gbufs[b], gsems[b]).wait()

        def fire_scatter(c, b):
            pltpu.async_copy(
                obufs[b].at[:, :, :, pl.ds(0, cols_w)],
                out_hbm.at[pl.ds(c * NB2, NB2), :, wid],
                ssems[b])

        def wait_scatter(b):
            pltpu.make_async_copy(
                obufs[b].at[:, :, :, pl.ds(0, cols_w)],
                out_hbm.at[pl.ds(0, NB2), :, 0],
                ssems[b]).wait()

        lane = lax.iota(jnp.int32, 16)
        dim_rows = [lane + 16 * h for h in range(DIM // 16)]
        big_rows = [lax.div(d, 8) for d in dim_rows]
        sub_rows = [lax.rem(d, 8) for d in dim_rows]
        s_ids = [jnp.broadcast_to(jnp.int32(s), (16,)) for s in range(NB2)]

        def scale(b):
            gbuf, obuf = gbufs[b], obufs[b]

            @plsc.parallel_loop(0, cols_w, step=1, unroll=4)
            def _scale(l):
                lcol = jnp.broadcast_to(l, (16,))
                for s in range(NB2):
                    for h in range(DIM // 16):
                        vec = gbuf[s * cols_w + l, pl.ds(16 * h, 16)]
                        plsc.store_scatter(
                            obuf, [s_ids[s], big_rows[h], sub_rows[h], lcol],
                            vec * SCALE)

        pltpu.sync_copy(xt_hbm.at[:, pl.ds(col0, cols_w)], idx_v)

        for c in range(GATHER_AHEAD):
            fire_gather(c, c % NBUF)

        @pl.loop(0, n_groups)
        def _group(g):
            for i in range(NBUF):
                c = g * NBUF + i
                wait_gather(i)

                @pl.when(c >= NBUF)
                def _():
                    wait_scatter(i)

                scale(i)
                fire_scatter(c, i)

                @pl.when(c + GATHER_AHEAD < n_chunks)
                def _():
                    fire_gather(c + GATHER_AHEAD, (i + GATHER_AHEAD) % NBUF)

        for c in range(n_chunks - NBUF, n_chunks):
            wait_scatter(c % NBUF)

    return emb_kernel


def kernel(x, table):
    B1, B2 = x.shape
    xt = jnp.transpose(x, (1, 0)).astype(jnp.int32)
    out5 = _make(B1, B2)(table, xt)
    out = jnp.transpose(out5, (0, 1, 3, 2, 4)).reshape(B2, DIM, B1)
    return jnp.transpose(out, (2, 0, 1))
